# baseline (speedup 1.0000x reference)
import jax
import jax.numpy as jnp
from jax import lax
from jax.experimental import pallas as pl
from jax.experimental.pallas import tpu as pltpu
from jax.experimental.pallas import tpu_sc as plsc

B = 16384
C = 26
EMB = 256
L = 16
NC = 2
NS = 16
NW = NC * NS
ROWS = B // NW
CH = 256


def _body(inp_hbm, out_hbm, inp_v, out_v):
    wid = lax.axis_index("s") * NC + lax.axis_index("c")
    base = wid * ROWS
    pltpu.sync_copy(inp_hbm.at[pl.ds(base, ROWS), :], inp_v)
    for c in range(ROWS // CH):
        def row_body(r, carry, c=c):
            head = inp_v[c * CH + r, pl.ds(0, L)]
            v = jnp.full((L,), head[0], jnp.int32)
            for j in range(EMB // L):
                out_v[r, pl.ds(j * L, L)] = v
            return carry
        lax.fori_loop(0, CH, row_body, 0)
        pltpu.sync_copy(out_v, out_hbm.at[pl.ds(base + c * CH, CH), :])


def kernel(inputs):
    mesh = plsc.VectorSubcoreMesh(core_axis_name="c", subcore_axis_name="s")
    k = pl.kernel(
        _body,
        out_type=jax.ShapeDtypeStruct((B, EMB), jnp.int32),
        mesh=mesh,
        scratch_types=[
            pltpu.VMEM((ROWS, C), jnp.int32),
            pltpu.VMEM((CH, EMB), jnp.int32),
        ],
    )
    return k(inputs)


# speedup vs baseline: 1.1020x; 1.1020x over previous
import jax
import jax.numpy as jnp
from jax import lax
from jax.experimental import pallas as pl
from jax.experimental.pallas import tpu as pltpu
from jax.experimental.pallas import tpu_sc as plsc

B = 16384
C = 26
EMB = 256
L = 16
NC = 2
NS = 16
NW = NC * NS
ROWS = B // NW
CH = 128
NCH = ROWS // CH


def _body(inp_hbm, out_hbm, inp_v, out0, out1, sem0, sem1):
    wid = lax.axis_index("s") * NC + lax.axis_index("c")
    base = wid * ROWS
    pltpu.sync_copy(inp_hbm.at[pl.ds(base, ROWS), :], inp_v)
    bufs = (out0, out1)
    sems = (sem0, sem1)
    pending = [None, None]
    for c in range(NCH):
        buf = bufs[c % 2]
        if pending[c % 2] is not None:
            pending[c % 2].wait()

        def row_body(r, carry, c=c, buf=buf):
            head = inp_v[c * CH + r, pl.ds(0, L)]
            v = jnp.full((L,), head[0], jnp.int32)
            for j in range(EMB // L):
                buf[r, pl.ds(j * L, L)] = v
            return carry

        lax.fori_loop(0, CH, row_body, 0)
        pending[c % 2] = pltpu.async_copy(
            buf, out_hbm.at[pl.ds(base + c * CH, CH), :], sems[c % 2]
        )
    for cp in pending:
        if cp is not None:
            cp.wait()


def kernel(inputs):
    mesh = plsc.VectorSubcoreMesh(core_axis_name="c", subcore_axis_name="s")
    k = pl.kernel(
        _body,
        out_type=jax.ShapeDtypeStruct((B, EMB), jnp.int32),
        mesh=mesh,
        scratch_types=[
            pltpu.VMEM((ROWS, C), jnp.int32),
            pltpu.VMEM((CH, EMB), jnp.int32),
            pltpu.VMEM((CH, EMB), jnp.int32),
            pltpu.SemaphoreType.DMA,
            pltpu.SemaphoreType.DMA,
        ],
    )
    return k(inputs)


# speedup vs baseline: 1.1449x; 1.0389x over previous
import jax
import jax.numpy as jnp
from jax import lax
from jax.experimental import pallas as pl
from jax.experimental.pallas import tpu as pltpu
from jax.experimental.pallas import tpu_sc as plsc

B = 16384
C = 26
EMB = 256
L = 16
NC = 2
NS = 16
NW = NC * NS
ROWS = B // NW
HALF = 128
CH = 256
NCH = ROWS // CH

def _body(inp_hbm, out_hbm, inp_v, buf0, buf1, sem0, sem1):
    wid = lax.axis_index("s") * NC + lax.axis_index("c")
    base = wid * ROWS
    pltpu.sync_copy(inp_hbm.at[pl.ds(base, ROWS), :], inp_v)
    bufs = (buf0, buf1)
    sems = (sem0, sem1)
    pending = [(), ()]
    for c in range(NCH):
        buf = bufs[c % 2]
        for cp in pending[c % 2]:
            cp.wait()

        def row_body(r, c=c, buf=buf):
            head = inp_v[c * CH + r, pl.ds(0, L)]
            splat = jnp.full((L,), head[0], jnp.int32)
            for j in range(HALF // L):
                buf[r, pl.ds(j * L, L)] = splat

        plsc.parallel_loop(0, CH, unroll=4)(row_body)

        pending[c % 2] = tuple(
            pltpu.async_copy(
                buf,
                out_hbm.at[pl.ds(base + c * CH, CH), pl.ds(h * HALF, HALF)],
                sems[c % 2],
            )
            for h in range(EMB // HALF)
        )
    for grp in pending:
        for cp in grp:
            cp.wait()


def kernel(inputs):
    mesh = plsc.VectorSubcoreMesh(core_axis_name="c", subcore_axis_name="s")
    k = pl.kernel(
        _body,
        out_type=jax.ShapeDtypeStruct((B, EMB), jnp.int32),
        mesh=mesh,
        scratch_types=[
            pltpu.VMEM((ROWS, C), jnp.int32),
            pltpu.VMEM((CH, HALF), jnp.int32),
            pltpu.VMEM((CH, HALF), jnp.int32),
            pltpu.SemaphoreType.DMA,
            pltpu.SemaphoreType.DMA,
        ],
    )
    return k(inputs)


# speedup vs baseline: 1.3444x; 1.1743x over previous
import jax
import jax.numpy as jnp
from jax import lax
from jax.experimental import pallas as pl
from jax.experimental.pallas import tpu as pltpu
from jax.experimental.pallas import tpu_sc as plsc

B = 16384
C = 26
EMB = 256
L = 16
NC = 2
NS = 16
NW = NC * NS
ROWS = B // NW
HALF = 128
CH = 256
NCH = ROWS // CH


def _body(xt_hbm, out_hbm, vals8, buf0, buf1, sem0, sem1):
    wid = lax.axis_index("s") * NC + lax.axis_index("c")
    base = wid * ROWS
    pltpu.sync_copy(xt_hbm.at[pl.ds(0, 8), pl.ds(base, ROWS)], vals8)
    bufs = (buf0, buf1)
    sems = (sem0, sem1)
    pending = [(), ()]
    for c in range(NCH):
        buf = bufs[c % 2]
        for cp in pending[c % 2]:
            cp.wait()

        def group_body(g, c=c, buf=buf):
            v16 = vals8[0, pl.ds(c * CH + g * L, L)]
            for i in range(L):
                splat = jnp.full((L,), v16[i], jnp.int32)
                for j in range(HALF // L):
                    buf[g * L + i, pl.ds(j * L, L)] = splat

        plsc.parallel_loop(0, CH // L, unroll=2)(group_body)

        pending[c % 2] = tuple(
            pltpu.async_copy(
                buf,
                out_hbm.at[pl.ds(base + c * CH, CH), pl.ds(h * HALF, HALF)],
                sems[c % 2],
            )
            for h in range(EMB // HALF)
        )
    for grp in pending:
        for cp in grp:
            cp.wait()


def kernel(inputs):
    xt = inputs.T
    mesh = plsc.VectorSubcoreMesh(core_axis_name="c", subcore_axis_name="s")
    k = pl.kernel(
        _body,
        out_type=jax.ShapeDtypeStruct((B, EMB), jnp.int32),
        mesh=mesh,
        scratch_types=[
            pltpu.VMEM((8, ROWS), jnp.int32),
            pltpu.VMEM((CH, HALF), jnp.int32),
            pltpu.VMEM((CH, HALF), jnp.int32),
            pltpu.SemaphoreType.DMA,
            pltpu.SemaphoreType.DMA,
        ],
    )
    return k(xt)
